# Initial kernel scaffold; baseline (speedup 1.0000x reference)
import jax
import jax.numpy as jnp
from jax.experimental import pallas as pl


def kernel(sphc, cg_rep, segment_ids):
    raise NotImplementedError("write your pallas kernel here")


# speedup vs baseline: 213.7832x; 213.7832x over previous
import jax
import jax.numpy as jnp
from jax.experimental import pallas as pl
from jax.experimental.pallas import tpu as pltpu

_NUM_SEG = 48


def _body(x_ref, s_ref, o_ref):
    x = x_ref[...]
    o_ref[...] = jnp.dot(x * x, s_ref[...], preferred_element_type=jnp.float32)


def kernel(sphc, cg_rep, segment_ids):
    B, M = sphc.shape
    seg = segment_ids.astype(jnp.int32)
    proj = (
        seg[:, None] == jnp.arange(_NUM_SEG, dtype=jnp.int32)[None, :]
    ).astype(jnp.float32) * cg_rep[:, None]
    TB = 1000
    grid = (pl.cdiv(B, TB),)
    out = pl.pallas_call(
        _body,
        grid=grid,
        in_specs=[
            pl.BlockSpec((TB, M), lambda i: (i, 0)),
            pl.BlockSpec((M, _NUM_SEG), lambda i: (0, 0)),
        ],
        out_specs=pl.BlockSpec((TB, _NUM_SEG), lambda i: (i, 0)),
        out_shape=jax.ShapeDtypeStruct((B, _NUM_SEG), jnp.float32),
        compiler_params=pltpu.CompilerParams(
            dimension_semantics=("arbitrary",),
        ),
    )(sphc, proj)
    return out


# speedup vs baseline: 266.5880x; 1.2470x over previous
import jax
import jax.numpy as jnp
from jax.experimental import pallas as pl
from jax.experimental.pallas import tpu as pltpu

_NUM_SEG = 48


def _body(x_ref, s_ref, o_ref):
    x = x_ref[...]
    o_ref[...] = jnp.dot(x * x, s_ref[...], preferred_element_type=jnp.float32)


def kernel(sphc, cg_rep, segment_ids):
    B, M = sphc.shape
    seg = segment_ids.astype(jnp.int32)
    proj = (
        seg[:, None] == jnp.arange(_NUM_SEG, dtype=jnp.int32)[None, :]
    ).astype(jnp.float32) * cg_rep[:, None]
    TB = 4000
    grid = (pl.cdiv(B, TB),)
    out = pl.pallas_call(
        _body,
        grid=grid,
        in_specs=[
            pl.BlockSpec((TB, M), lambda i: (i, 0)),
            pl.BlockSpec((M, _NUM_SEG), lambda i: (0, 0)),
        ],
        out_specs=pl.BlockSpec((TB, _NUM_SEG), lambda i: (i, 0)),
        out_shape=jax.ShapeDtypeStruct((B, _NUM_SEG), jnp.float32),
        compiler_params=pltpu.CompilerParams(
            dimension_semantics=("arbitrary",),
        ),
    )(sphc, proj)
    return out
